# baseline (speedup 1.0000x reference)
import functools

import numpy as np
import jax
import jax.numpy as jnp
from jax.experimental import pallas as pl
from jax.experimental.pallas import tpu as pltpu

_S = 0.025
_ALPHA = 0.98
_DELTA = 2.0
_EPS = 1e-6

_L = 128

_IDX = np.arange(_L)
_DIFF = _IDX[None, :] - _IDX[:, None]
_W_NP = np.where(_DIFF >= 0,
                 _S * (1.0 - _S) ** np.maximum(_DIFF, 0),
                 0.0).astype(np.float32)
_D_NP = np.zeros((_L, _L), np.float32)
_D_NP[_L - 1, :] = (1.0 - _S) ** (_IDX + 1.0)
_WD_NP = np.concatenate([_W_NP, _D_NP], axis=0)
_SQRT_DELTA = float(np.sqrt(_DELTA))


_G = 4


def _pcen_body(x_ref, wd_ref, o_ref, *, t_total, n_t, c_rows):
    wd = wd_ref[...]
    rows = _G * c_rows
    mp = jnp.zeros((rows, _L), jnp.float32)
    for t in range(n_t):
        lo, hi = t * _L, (t + 1) * _L
        tail = hi > t_total
        if tail:
            w_valid = t_total - lo
            xc = x_ref[:, :, lo:t_total].reshape(rows, w_valid)
            xc = jnp.concatenate(
                [xc, jnp.zeros((rows, _L - w_valid), jnp.float32)], axis=1)
        else:
            xc = x_ref[:, :, lo:hi].reshape(rows, _L)
        z = jnp.concatenate([xc, mp], axis=1)
        m = jnp.dot(z, wd, preferred_element_type=jnp.float32)
        mp = m
        p = jnp.exp(-_ALPHA * jnp.log(_EPS + m))
        y = jnp.sqrt(xc * p + _DELTA) - _SQRT_DELTA
        if tail:
            o_ref[:, :, lo:t_total] = y[:, :w_valid].reshape(_G, c_rows, w_valid)
        else:
            o_ref[:, :, lo:hi] = y.reshape(_G, c_rows, _L)


def kernel(mel_power):
    B, C, T = mel_power.shape
    n_t = pl.cdiv(T, _L)
    out = pl.pallas_call(
        functools.partial(_pcen_body, t_total=T, n_t=n_t, c_rows=C),
        grid=(B // _G,),
        in_specs=[
            pl.BlockSpec((_G, C, T), lambda b: (b, 0, 0)),
            pl.BlockSpec((2 * _L, _L), lambda b: (0, 0)),
        ],
        out_specs=pl.BlockSpec((_G, C, T), lambda b: (b, 0, 0)),
        out_shape=jax.ShapeDtypeStruct((B, C, T), jnp.float32),
        compiler_params=pltpu.CompilerParams(
            dimension_semantics=("parallel",),
            vmem_limit_bytes=100 * 1024 * 1024,
        ),
    )(mel_power, jnp.asarray(_WD_NP))
    return out


# speedup vs baseline: 1.0125x; 1.0125x over previous
import functools

import numpy as np
import jax
import jax.numpy as jnp
from jax.experimental import pallas as pl
from jax.experimental.pallas import tpu as pltpu

_S = 0.025
_ALPHA = 0.98
_DELTA = 2.0
_EPS = 1e-6

_L = 128

_IDX = np.arange(_L)
_DIFF = _IDX[None, :] - _IDX[:, None]
_W_NP = np.where(_DIFF >= 0,
                 _S * (1.0 - _S) ** np.maximum(_DIFF, 0),
                 0.0).astype(np.float32)
_D_NP = np.zeros((_L, _L), np.float32)
_D_NP[_L - 1, :] = (1.0 - _S) ** (_IDX + 1.0)
_WD_NP = np.concatenate([_W_NP, _D_NP], axis=0)
_SQRT_DELTA = float(np.sqrt(_DELTA))


_G = 4


def _pcen_body(x_ref, wd_ref, o_ref, *, t_total, n_t, c_rows):
    wd = wd_ref[...]
    rows = _G * c_rows
    mp = jnp.zeros((rows, _L), jnp.float32)
    for t in range(n_t):
        lo, hi = t * _L, (t + 1) * _L
        tail = hi > t_total
        if tail:
            w_valid = t_total - lo
            xc = x_ref[:, :, lo:t_total].reshape(rows, w_valid)
            xc = jnp.concatenate(
                [xc, jnp.zeros((rows, _L - w_valid), jnp.float32)], axis=1)
        else:
            xc = x_ref[:, :, lo:hi].reshape(rows, _L)
        z = jnp.concatenate([xc, mp], axis=1)
        m = jnp.dot(z, wd, preferred_element_type=jnp.float32)
        mp = m
        p = jnp.exp2(-_ALPHA * jnp.log2(_EPS + m))
        u = xc * p + _DELTA
        y = u * jax.lax.rsqrt(u) - _SQRT_DELTA
        if tail:
            o_ref[:, :, lo:t_total] = y[:, :w_valid].reshape(_G, c_rows, w_valid)
        else:
            o_ref[:, :, lo:hi] = y.reshape(_G, c_rows, _L)


def kernel(mel_power):
    B, C, T = mel_power.shape
    n_t = pl.cdiv(T, _L)
    out = pl.pallas_call(
        functools.partial(_pcen_body, t_total=T, n_t=n_t, c_rows=C),
        grid=(B // _G,),
        in_specs=[
            pl.BlockSpec((_G, C, T), lambda b: (b, 0, 0)),
            pl.BlockSpec((2 * _L, _L), lambda b: (0, 0)),
        ],
        out_specs=pl.BlockSpec((_G, C, T), lambda b: (b, 0, 0)),
        out_shape=jax.ShapeDtypeStruct((B, C, T), jnp.float32),
        compiler_params=pltpu.CompilerParams(
            dimension_semantics=("parallel",),
            vmem_limit_bytes=100 * 1024 * 1024,
        ),
    )(mel_power, jnp.asarray(_WD_NP))
    return out


# speedup vs baseline: 3.1304x; 3.0916x over previous
import functools

import numpy as np
import jax
import jax.numpy as jnp
from jax.experimental import pallas as pl
from jax.experimental.pallas import tpu as pltpu

_S = 0.025
_ALPHA = 0.98
_DELTA = 2.0
_EPS = 1e-6

_L = 128
_G = 4

_IDX = np.arange(_L)
_DIFF = _IDX[:, None] - _IDX[None, :]
_W_NP = np.where(_DIFF >= 0,
                 _S * (1.0 - _S) ** np.maximum(_DIFF, 0),
                 0.0).astype(np.float32)
_D_NP = np.zeros((_L, _L), np.float32)
_D_NP[:, _L - 1] = (1.0 - _S) ** (_IDX + 1.0)
_WD_NP = np.concatenate([_W_NP, _D_NP], axis=1)
_SQRT_DELTA = float(np.sqrt(_DELTA))


def _pcen_body(x_ref, wd_ref, o_ref, *, t_total, n_t, c):
    wd = wd_ref[...]
    mp = [jnp.zeros((_L, c), jnp.float32) for _ in range(_G)]
    for t in range(n_t):
        lo, hi = t * _L, (t + 1) * _L
        tail = hi > t_total
        w_valid = t_total - lo if tail else _L
        for g in range(_G):
            xc = x_ref[g, lo:lo + w_valid, :]
            if tail:
                xc = jnp.concatenate(
                    [xc, jnp.zeros((_L - w_valid, c), jnp.float32)], axis=0)
            z = jnp.concatenate([xc, mp[g]], axis=0)
            m = jnp.dot(wd, z, preferred_element_type=jnp.float32)
            mp[g] = m
            p = jnp.exp2(-_ALPHA * jnp.log2(_EPS + m))
            u = xc * p + _DELTA
            y = u * jax.lax.rsqrt(u) - _SQRT_DELTA
            o_ref[g, lo:lo + w_valid, :] = y[:w_valid]


def kernel(mel_power):
    B, C, T = mel_power.shape
    xt = jnp.swapaxes(mel_power, 1, 2)
    n_t = pl.cdiv(T, _L)
    out = pl.pallas_call(
        functools.partial(_pcen_body, t_total=T, n_t=n_t, c=C),
        grid=(B // _G,),
        in_specs=[
            pl.BlockSpec((_G, T, C), lambda b: (b, 0, 0)),
            pl.BlockSpec((_L, 2 * _L), lambda b: (0, 0)),
        ],
        out_specs=pl.BlockSpec((_G, T, C), lambda b: (b, 0, 0)),
        out_shape=jax.ShapeDtypeStruct((B, T, C), jnp.float32),
        compiler_params=pltpu.CompilerParams(
            dimension_semantics=("parallel",),
            vmem_limit_bytes=100 * 1024 * 1024,
        ),
    )(xt, jnp.asarray(_WD_NP))
    return jnp.swapaxes(out, 1, 2)


# speedup vs baseline: 3.5170x; 1.1235x over previous
import functools

import numpy as np
import jax
import jax.numpy as jnp
from jax.experimental import pallas as pl
from jax.experimental.pallas import tpu as pltpu

_S = 0.025
_ALPHA = 0.98
_DELTA = 2.0
_EPS = 1e-6

_L = 128
_G = 8

_IDX = np.arange(_L)
_DIFF = _IDX[:, None] - _IDX[None, :]
_W_NP = np.where(_DIFF >= 0,
                 _S * (1.0 - _S) ** np.maximum(_DIFF, 0),
                 0.0).astype(np.float32)
_D_NP = np.zeros((_L, _L), np.float32)
_D_NP[:, _L - 1] = (1.0 - _S) ** (_IDX + 1.0)
_WD_NP = np.concatenate([_W_NP, _D_NP], axis=1)
_SQRT_DELTA = float(np.sqrt(_DELTA))


def _pcen_body(x_ref, wd_ref, o_ref, *, t_total, n_t, c):
    wd = wd_ref[...]
    mp = [jnp.zeros((_L, c), jnp.float32) for _ in range(_G)]
    for t in range(n_t):
        lo, hi = t * _L, (t + 1) * _L
        tail = hi > t_total
        w_valid = t_total - lo if tail else _L
        for g in range(_G):
            xc = x_ref[g, lo:lo + w_valid, :]
            if tail:
                xc = jnp.concatenate(
                    [xc, jnp.zeros((_L - w_valid, c), jnp.float32)], axis=0)
            z = jnp.concatenate([xc, mp[g]], axis=0)
            m = jnp.dot(wd, z, preferred_element_type=jnp.float32)
            mp[g] = m
            p = jnp.exp2(-_ALPHA * jnp.log2(_EPS + m))
            u = xc * p + _DELTA
            y = u * jax.lax.rsqrt(u) - _SQRT_DELTA
            o_ref[g, lo:lo + w_valid, :] = y[:w_valid]


def kernel(mel_power):
    B, C, T = mel_power.shape
    xt = jnp.swapaxes(mel_power, 1, 2)
    n_t = pl.cdiv(T, _L)
    out = pl.pallas_call(
        functools.partial(_pcen_body, t_total=T, n_t=n_t, c=C),
        grid=(B // _G,),
        in_specs=[
            pl.BlockSpec((_G, T, C), lambda b: (b, 0, 0)),
            pl.BlockSpec((_L, 2 * _L), lambda b: (0, 0)),
        ],
        out_specs=pl.BlockSpec((_G, T, C), lambda b: (b, 0, 0)),
        out_shape=jax.ShapeDtypeStruct((B, T, C), jnp.float32),
        compiler_params=pltpu.CompilerParams(
            dimension_semantics=("parallel",),
            vmem_limit_bytes=100 * 1024 * 1024,
        ),
    )(xt, jnp.asarray(_WD_NP))
    return jnp.swapaxes(out, 1, 2)
